# baseline (speedup 1.0000x reference)
import functools

import jax
import jax.numpy as jnp
from jax import lax
from jax.experimental import pallas as pl
from jax.experimental.pallas import tpu as pltpu
from jax.experimental.pallas import tpu_sc as plsc

N = 1_000_000
NW = 32
PER_W = 31_232
MAIN = NW * PER_W
TAIL = N - MAIN
KCH = 4
CHUNK = PER_W // KCH
UNROLL = 8
IN_IT = CHUNK // 16
NBINS = 64
ROW = 256
LN2 = 0.69314718055994530942
SQRT2 = 1.4142135623730951


def _vlog(x):
    bits = plsc.bitcast(x, jnp.int32)
    e = (bits >> 23) - 127
    m = plsc.bitcast((bits & 0x007FFFFF) | 0x3F800000, jnp.float32)
    big = m > SQRT2
    m = jnp.where(big, m * 0.5, m)
    ef = (e + big.astype(jnp.int32)).astype(jnp.float32)
    s = (m - 1.0) / (m + 1.0)
    t = s * s
    poly = 1.0 + t * (1.0 / 3.0 + t * (1.0 / 5.0 + t * (1.0 / 7.0 + t * (1.0 / 9.0))))
    return ef * LN2 + 2.0 * s * poly


def _sc_body(conf_hbm, lf_hbm, tgt_hbm, bnd_hbm, out_hbm,
             cbuf, b0, b1, b2, tbuf, ec, e0, e1, e2, et, bnd, hc, hs, hr, acc, prow):
    nc = 2
    wid = lax.axis_index("s") * nc + lax.axis_index("c")
    base = wid * PER_W
    lane = lax.iota(jnp.int32, 16)
    zero16 = jnp.zeros((16,), jnp.float32)

    pltpu.sync_copy(bnd_hbm, bnd)
    for i in range(NBINS):
        hc[pl.ds(i * 16, 16)] = zero16
        hs[pl.ds(i * 16, 16)] = zero16
        hr[pl.ds(i * 16, 16)] = zero16

    def sample16(i, cb, lb0, lb1, lb2, tb):
        conf = cb[pl.ds(i * 16, 16)]
        tgt = tb[pl.ds(i * 16, 16)]
        l0 = lb0[pl.ds(i * 16, 16)]
        l1 = lb1[pl.ds(i * 16, 16)]
        l2 = lb2[pl.ds(i * 16, 16)]
        pred = jnp.where(l2 > jnp.maximum(l0, l1), 2, jnp.where(l1 > l0, 1, 0))
        corr = (pred == tgt).astype(jnp.float32)
        p = jnp.maximum(conf, 1e-12)
        q = jnp.where(corr > 0.5, p, 1.0 - p)
        y = conf * 50.0
        iy = y.astype(jnp.int32)
        j = iy + (y > iy.astype(jnp.float32)).astype(jnp.int32) - 1
        jc = jnp.clip(j, 0, 49)
        lo = plsc.load_gather(bnd, [jc])
        hi = plsc.load_gather(bnd, [jc + 1])
        j2 = jc + (conf > hi).astype(jnp.int32) - (conf <= lo).astype(jnp.int32)
        valid = j2 >= 0
        hidx = lane * NBINS + jnp.maximum(j2, 0)
        plsc.addupdate_scatter(hc, [hidx], jnp.ones((16,), jnp.float32), mask=valid)
        plsc.addupdate_scatter(hs, [hidx], conf, mask=valid)
        plsc.addupdate_scatter(hr, [hidx], corr, mask=valid)
        return -_vlog(q)

    def chunk_body(g, a):
        off = base + g * CHUNK
        pltpu.sync_copy(conf_hbm.at[0, pl.ds(off, CHUNK)], cbuf)
        pltpu.sync_copy(lf_hbm.at[pl.ds(off, CHUNK)], b0)
        pltpu.sync_copy(lf_hbm.at[pl.ds(N + off, CHUNK)], b1)
        pltpu.sync_copy(lf_hbm.at[pl.ds(2 * N + off, CHUNK)], b2)
        pltpu.sync_copy(tgt_hbm.at[pl.ds(off, CHUNK)], tbuf)

        def inner(i, a2):
            for u in range(UNROLL):
                a2 = a2 + sample16(i * UNROLL + u, cbuf, b0, b1, b2, tbuf)
            return a2

        return lax.fori_loop(0, IN_IT // UNROLL, inner, a)

    acc_main = lax.fori_loop(0, KCH, chunk_body, zero16)
    acc[...] = zero16

    @pl.when(wid == 0)
    def _tail():
        pltpu.sync_copy(conf_hbm.at[0, pl.ds(MAIN, TAIL)], ec)
        pltpu.sync_copy(lf_hbm.at[pl.ds(MAIN, TAIL)], e0)
        pltpu.sync_copy(lf_hbm.at[pl.ds(N + MAIN, TAIL)], e1)
        pltpu.sync_copy(lf_hbm.at[pl.ds(2 * N + MAIN, TAIL)], e2)
        pltpu.sync_copy(tgt_hbm.at[pl.ds(MAIN, TAIL)], et)
        a2 = zero16
        for i in range(TAIL // 16):
            a2 = a2 + sample16(i, ec, e0, e1, e2, et)
        acc[...] = a2

    for g in range(4):
        vc = zero16
        vs = zero16
        vr = zero16
        for l in range(16):
            o = l * NBINS + g * 16
            vc = vc + hc[pl.ds(o, 16)]
            vs = vs + hs[pl.ds(o, 16)]
            vr = vr + hr[pl.ds(o, 16)]
        prow[pl.ds(g * 16, 16)] = vc
        prow[pl.ds(64 + g * 16, 16)] = vs
        prow[pl.ds(128 + g * 16, 16)] = vr
    prow[pl.ds(192, 16)] = acc_main + acc[...]
    prow[pl.ds(208, 16)] = zero16
    prow[pl.ds(224, 16)] = zero16
    prow[pl.ds(240, 16)] = zero16
    pltpu.sync_copy(prow, out_hbm.at[pl.ds(wid * ROW, ROW)])


_sc_hist = functools.partial(
    pl.kernel,
    out_type=jax.ShapeDtypeStruct((NW * ROW,), jnp.float32),
    mesh=plsc.VectorSubcoreMesh(core_axis_name="c", subcore_axis_name="s"),
    compiler_params=pltpu.CompilerParams(needs_layout_passes=False),
    scratch_types=[
        pltpu.VMEM((CHUNK,), jnp.float32),
        pltpu.VMEM((CHUNK,), jnp.float32),
        pltpu.VMEM((CHUNK,), jnp.float32),
        pltpu.VMEM((CHUNK,), jnp.float32),
        pltpu.VMEM((CHUNK,), jnp.int32),
        pltpu.VMEM((TAIL,), jnp.float32),
        pltpu.VMEM((TAIL,), jnp.float32),
        pltpu.VMEM((TAIL,), jnp.float32),
        pltpu.VMEM((TAIL,), jnp.float32),
        pltpu.VMEM((TAIL,), jnp.int32),
        pltpu.VMEM((56,), jnp.float32),
        pltpu.VMEM((16 * NBINS,), jnp.float32),
        pltpu.VMEM((16 * NBINS,), jnp.float32),
        pltpu.VMEM((16 * NBINS,), jnp.float32),
        pltpu.VMEM((16,), jnp.float32),
        pltpu.VMEM((ROW,), jnp.float32),
    ],
)(_sc_body)


def _combine(x_ref, t_ref, b_ref, e_ref):
    x = x_ref[...]
    nf = jnp.float32(N)
    cnt = jnp.sum(x[:, 0:64], axis=0, keepdims=True)
    sconf = jnp.sum(x[:, 64:128], axis=0, keepdims=True)
    scorr = jnp.sum(x[:, 128:192], axis=0, keepdims=True)
    bce = jnp.sum(x[:, 192:208]) / nf
    safe = jnp.maximum(cnt, 1.0)
    term = jnp.where(cnt > 0, (cnt / nf) * jnp.abs(scorr / safe - sconf / safe), 0.0)
    ece = jnp.sum(term)
    t_ref[0, 0] = bce + ece
    b_ref[0, 0] = bce
    e_ref[0, 0] = ece


def kernel(confidence, direction_logits, targets):
    conf = confidence.T
    lflat = direction_logits.T.reshape(3 * N)
    bounds = jnp.pad(jnp.linspace(0.0, 1.0, 51), (0, 5), constant_values=2.0)
    partial = _sc_hist(conf, lflat, targets, bounds)
    total, bce, ece = pl.pallas_call(
        _combine,
        out_shape=(
            jax.ShapeDtypeStruct((1, 1), jnp.float32),
            jax.ShapeDtypeStruct((1, 1), jnp.float32),
            jax.ShapeDtypeStruct((1, 1), jnp.float32),
        ),
        out_specs=(
            pl.BlockSpec(memory_space=pltpu.SMEM),
            pl.BlockSpec(memory_space=pltpu.SMEM),
            pl.BlockSpec(memory_space=pltpu.SMEM),
        ),
    )(partial.reshape(NW, ROW))
    return (total[0, 0], bce[0, 0], ece[0, 0])


# speedup vs baseline: 1.1117x; 1.1117x over previous
import functools

import jax
import jax.numpy as jnp
from jax import lax
from jax.experimental import pallas as pl
from jax.experimental.pallas import tpu as pltpu
from jax.experimental.pallas import tpu_sc as plsc

N = 1_000_000
NW = 32
PER_W = 31_232
MAIN = NW * PER_W
TAIL = N - MAIN
KCH = 4
CHUNK = PER_W // KCH
UNROLL = 4
IN_IT = CHUNK // 16
NBINS = 64
ROW = 256
LN2 = 0.69314718055994530942
SQRT2 = 1.4142135623730951


def _vlog(x):
    bits = plsc.bitcast(x, jnp.int32)
    e = (bits >> 23) - 127
    m = plsc.bitcast((bits & 0x007FFFFF) | 0x3F800000, jnp.float32)
    big = m > SQRT2
    m = jnp.where(big, m * 0.5, m)
    ef = (e + big.astype(jnp.int32)).astype(jnp.float32)
    s = (m - 1.0) / (m + 1.0)
    t = s * s
    poly = 1.0 + t * (1.0 / 3.0 + t * (1.0 / 5.0 + t * (1.0 / 7.0 + t * (1.0 / 9.0))))
    return ef * LN2 + 2.0 * s * poly


def _sc_body(conf_hbm, lf_hbm, tgt_hbm, bnd_hbm, out_hbm,
             cbufA, b0A, b1A, b2A, tbufA, cbufB, b0B, b1B, b2B, tbufB,
             ec, e0, e1, e2, et, bnd, hc, hs, hr, acc, prow, semA, semB):
    nc = 2
    wid = lax.axis_index("s") * nc + lax.axis_index("c")
    base = wid * PER_W
    lane = lax.iota(jnp.int32, 16)
    zero16 = jnp.zeros((16,), jnp.float32)

    pltpu.sync_copy(bnd_hbm, bnd)
    for i in range(NBINS):
        hc[pl.ds(i * 16, 16)] = zero16
        hs[pl.ds(i * 16, 16)] = zero16
        hr[pl.ds(i * 16, 16)] = zero16

    def sample16(i, cb, lb0, lb1, lb2, tb):
        conf = cb[pl.ds(i * 16, 16)]
        tgt = tb[pl.ds(i * 16, 16)]
        l0 = lb0[pl.ds(i * 16, 16)]
        l1 = lb1[pl.ds(i * 16, 16)]
        l2 = lb2[pl.ds(i * 16, 16)]
        pred = jnp.where(l2 > jnp.maximum(l0, l1), 2, jnp.where(l1 > l0, 1, 0))
        corr = (pred == tgt).astype(jnp.float32)
        p = jnp.maximum(conf, 1e-12)
        q = jnp.where(corr > 0.5, p, 1.0 - p)
        y = conf * 50.0
        iy = y.astype(jnp.int32)
        j = iy + (y > iy.astype(jnp.float32)).astype(jnp.int32) - 1
        jc = jnp.clip(j, 0, 49)
        lo = plsc.load_gather(bnd, [jc])
        hi = plsc.load_gather(bnd, [jc + 1])
        j2 = jc + (conf > hi).astype(jnp.int32) - (conf <= lo).astype(jnp.int32)
        valid = j2 >= 0
        hidx = lane * NBINS + jnp.maximum(j2, 0)
        plsc.addupdate_scatter(hc, [hidx], jnp.ones((16,), jnp.float32), mask=valid)
        plsc.addupdate_scatter(hs, [hidx], conf, mask=valid)
        plsc.addupdate_scatter(hr, [hidx], corr, mask=valid)
        return -_vlog(q)

    bufs = [(cbufA, b0A, b1A, b2A, tbufA, semA), (cbufB, b0B, b1B, b2B, tbufB, semB)]

    def start_chunk(g):
        cb, x0, x1, x2, tb, sem = bufs[g % 2]
        off = base + g * CHUNK
        copies = (
            pltpu.async_copy(conf_hbm.at[0, pl.ds(off, CHUNK)], cb, sem),
            pltpu.async_copy(lf_hbm.at[pl.ds(off, CHUNK)], x0, sem),
            pltpu.async_copy(lf_hbm.at[pl.ds(N + off, CHUNK)], x1, sem),
            pltpu.async_copy(lf_hbm.at[pl.ds(2 * N + off, CHUNK)], x2, sem),
            pltpu.async_copy(tgt_hbm.at[pl.ds(off, CHUNK)], tb, sem),
        )
        return copies

    def compute_chunk(copies, g, a):
        cb, x0, x1, x2, tb, sem = bufs[g % 2]
        for cp in copies:
            cp.wait()

        def inner(i, a2):
            for u in range(UNROLL):
                a2 = a2 + sample16(i * UNROLL + u, cb, x0, x1, x2, tb)
            return a2

        return lax.fori_loop(0, IN_IT // UNROLL, inner, a)

    acc_main = zero16
    pending = start_chunk(0)
    for g in range(KCH):
        nxt = start_chunk(g + 1) if g + 1 < KCH else None
        acc_main = compute_chunk(pending, g, acc_main)
        pending = nxt
    acc[...] = zero16

    @pl.when(wid == 0)
    def _tail():
        pltpu.sync_copy(conf_hbm.at[0, pl.ds(MAIN, TAIL)], ec)
        pltpu.sync_copy(lf_hbm.at[pl.ds(MAIN, TAIL)], e0)
        pltpu.sync_copy(lf_hbm.at[pl.ds(N + MAIN, TAIL)], e1)
        pltpu.sync_copy(lf_hbm.at[pl.ds(2 * N + MAIN, TAIL)], e2)
        pltpu.sync_copy(tgt_hbm.at[pl.ds(MAIN, TAIL)], et)
        a2 = zero16
        for i in range(TAIL // 16):
            a2 = a2 + sample16(i, ec, e0, e1, e2, et)
        acc[...] = a2

    for g in range(4):
        vc = zero16
        vs = zero16
        vr = zero16
        for l in range(16):
            o = l * NBINS + g * 16
            vc = vc + hc[pl.ds(o, 16)]
            vs = vs + hs[pl.ds(o, 16)]
            vr = vr + hr[pl.ds(o, 16)]
        prow[pl.ds(g * 16, 16)] = vc
        prow[pl.ds(64 + g * 16, 16)] = vs
        prow[pl.ds(128 + g * 16, 16)] = vr
    prow[pl.ds(192, 16)] = acc_main + acc[...]
    prow[pl.ds(208, 16)] = zero16
    prow[pl.ds(224, 16)] = zero16
    prow[pl.ds(240, 16)] = zero16
    pltpu.sync_copy(prow, out_hbm.at[pl.ds(wid * ROW, ROW)])


_sc_hist = functools.partial(
    pl.kernel,
    out_type=jax.ShapeDtypeStruct((NW * ROW,), jnp.float32),
    mesh=plsc.VectorSubcoreMesh(core_axis_name="c", subcore_axis_name="s"),
    compiler_params=pltpu.CompilerParams(needs_layout_passes=False),
    scratch_types=[
        pltpu.VMEM((CHUNK,), jnp.float32),
        pltpu.VMEM((CHUNK,), jnp.float32),
        pltpu.VMEM((CHUNK,), jnp.float32),
        pltpu.VMEM((CHUNK,), jnp.float32),
        pltpu.VMEM((CHUNK,), jnp.int32),
        pltpu.VMEM((CHUNK,), jnp.float32),
        pltpu.VMEM((CHUNK,), jnp.float32),
        pltpu.VMEM((CHUNK,), jnp.float32),
        pltpu.VMEM((CHUNK,), jnp.float32),
        pltpu.VMEM((CHUNK,), jnp.int32),
        pltpu.VMEM((TAIL,), jnp.float32),
        pltpu.VMEM((TAIL,), jnp.float32),
        pltpu.VMEM((TAIL,), jnp.float32),
        pltpu.VMEM((TAIL,), jnp.float32),
        pltpu.VMEM((TAIL,), jnp.int32),
        pltpu.VMEM((56,), jnp.float32),
        pltpu.VMEM((16 * NBINS,), jnp.float32),
        pltpu.VMEM((16 * NBINS,), jnp.float32),
        pltpu.VMEM((16 * NBINS,), jnp.float32),
        pltpu.VMEM((16,), jnp.float32),
        pltpu.VMEM((ROW,), jnp.float32),
        pltpu.SemaphoreType.DMA,
        pltpu.SemaphoreType.DMA,
    ],
)(_sc_body)


def _combine(x_ref, t_ref, b_ref, e_ref):
    x = x_ref[...]
    nf = jnp.float32(N)
    cnt = jnp.sum(x[:, 0:64], axis=0, keepdims=True)
    sconf = jnp.sum(x[:, 64:128], axis=0, keepdims=True)
    scorr = jnp.sum(x[:, 128:192], axis=0, keepdims=True)
    bce = jnp.sum(x[:, 192:208]) / nf
    safe = jnp.maximum(cnt, 1.0)
    term = jnp.where(cnt > 0, (cnt / nf) * jnp.abs(scorr / safe - sconf / safe), 0.0)
    ece = jnp.sum(term)
    t_ref[0, 0] = bce + ece
    b_ref[0, 0] = bce
    e_ref[0, 0] = ece


def kernel(confidence, direction_logits, targets):
    conf = confidence.T
    lflat = direction_logits.T.reshape(3 * N)
    bounds = jnp.pad(jnp.linspace(0.0, 1.0, 51), (0, 5), constant_values=2.0)
    partial = _sc_hist(conf, lflat, targets, bounds)
    total, bce, ece = pl.pallas_call(
        _combine,
        out_shape=(
            jax.ShapeDtypeStruct((1, 1), jnp.float32),
            jax.ShapeDtypeStruct((1, 1), jnp.float32),
            jax.ShapeDtypeStruct((1, 1), jnp.float32),
        ),
        out_specs=(
            pl.BlockSpec(memory_space=pltpu.SMEM),
            pl.BlockSpec(memory_space=pltpu.SMEM),
            pl.BlockSpec(memory_space=pltpu.SMEM),
        ),
    )(partial.reshape(NW, ROW))
    return (total[0, 0], bce[0, 0], ece[0, 0])


# speedup vs baseline: 1.1563x; 1.0401x over previous
import functools

import jax
import jax.numpy as jnp
from jax import lax
from jax.experimental import pallas as pl
from jax.experimental.pallas import tpu as pltpu
from jax.experimental.pallas import tpu_sc as plsc

N = 1_000_000
NW = 32
PER_W = 31_232
MAIN = NW * PER_W
TAIL = N - MAIN
KCH = 4
CHUNK = PER_W // KCH
UNROLL = 4
IN_IT = CHUNK // 16
NBINS = 64
ROW = 256
LN2 = 0.69314718055994530942
SQRT2 = 1.4142135623730951


def _vlog(x):
    bits = plsc.bitcast(x, jnp.int32)
    e = (bits >> 23) - 127
    m = plsc.bitcast((bits & 0x007FFFFF) | 0x3F800000, jnp.float32)
    ef = e.astype(jnp.float32)
    s = (m - 1.0) / (m + 1.0)
    t = s * s
    poly = 1.0 / 3.0 + t * (1.0 / 5.0 + t * (1.0 / 7.0 + t * (1.0 / 9.0)))
    return ef * LN2 + 2.0 * s * (1.0 + t * poly)


def _sc_body(conf_hbm, lf_hbm, tgt_hbm, bnd_hbm, out_hbm,
             cbufA, b0A, b1A, b2A, tbufA, cbufB, b0B, b1B, b2B, tbufB,
             ec, e0, e1, e2, et, bnd, hs, hr, acc, prow, semA, semB):
    nc = 2
    wid = lax.axis_index("s") * nc + lax.axis_index("c")
    base = wid * PER_W
    lane = lax.iota(jnp.int32, 16)
    zero16 = jnp.zeros((16,), jnp.float32)

    pltpu.sync_copy(bnd_hbm, bnd)
    for i in range(NBINS):
        hs[pl.ds(i * 16, 16)] = zero16
        hr[pl.ds(i * 16, 16)] = zero16

    def sample16(i, cb, lb0, lb1, lb2, tb):
        conf = cb[pl.ds(i * 16, 16)]
        tgt = tb[pl.ds(i * 16, 16)]
        l0 = lb0[pl.ds(i * 16, 16)]
        l1 = lb1[pl.ds(i * 16, 16)]
        l2 = lb2[pl.ds(i * 16, 16)]
        pred = jnp.where(l2 > jnp.maximum(l0, l1), 2, jnp.where(l1 > l0, 1, 0))
        corr = (pred == tgt).astype(jnp.float32)
        p = jnp.maximum(conf, 1e-12)
        q = jnp.where(corr > 0.5, p, 1.0 - p)
        y = conf * 50.0
        iy = y.astype(jnp.int32)
        j = iy + (y > iy.astype(jnp.float32)).astype(jnp.int32) - 1
        jc = jnp.clip(j, 0, 49)
        lo = plsc.load_gather(bnd, [jc])
        hi = plsc.load_gather(bnd, [jc + 1])
        j2 = jc + (conf > hi).astype(jnp.int32) - (conf <= lo).astype(jnp.int32)
        valid = j2 >= 0
        hidx = lane * NBINS + jnp.maximum(j2, 0)
        plsc.addupdate_scatter(hs, [hidx], conf, mask=valid)
        plsc.addupdate_scatter(hr, [hidx], corr + 4096.0, mask=valid)
        return -_vlog(q)

    bufs = [(cbufA, b0A, b1A, b2A, tbufA, semA), (cbufB, b0B, b1B, b2B, tbufB, semB)]

    def start_chunk(g):
        cb, x0, x1, x2, tb, sem = bufs[g % 2]
        off = base + g * CHUNK
        copies = (
            pltpu.async_copy(conf_hbm.at[0, pl.ds(off, CHUNK)], cb, sem),
            pltpu.async_copy(lf_hbm.at[pl.ds(off, CHUNK)], x0, sem),
            pltpu.async_copy(lf_hbm.at[pl.ds(N + off, CHUNK)], x1, sem),
            pltpu.async_copy(lf_hbm.at[pl.ds(2 * N + off, CHUNK)], x2, sem),
            pltpu.async_copy(tgt_hbm.at[pl.ds(off, CHUNK)], tb, sem),
        )
        return copies

    def compute_chunk(copies, g, a):
        cb, x0, x1, x2, tb, sem = bufs[g % 2]
        for cp in copies:
            cp.wait()

        def inner(i, a2):
            for u in range(UNROLL):
                a2 = a2 + sample16(i * UNROLL + u, cb, x0, x1, x2, tb)
            return a2

        return lax.fori_loop(0, IN_IT // UNROLL, inner, a)

    acc_main = zero16
    pending = start_chunk(0)
    for g in range(KCH):
        nxt = start_chunk(g + 1) if g + 1 < KCH else None
        acc_main = compute_chunk(pending, g, acc_main)
        pending = nxt
    acc[...] = zero16

    @pl.when(wid == 0)
    def _tail():
        pltpu.sync_copy(conf_hbm.at[0, pl.ds(MAIN, TAIL)], ec)
        pltpu.sync_copy(lf_hbm.at[pl.ds(MAIN, TAIL)], e0)
        pltpu.sync_copy(lf_hbm.at[pl.ds(N + MAIN, TAIL)], e1)
        pltpu.sync_copy(lf_hbm.at[pl.ds(2 * N + MAIN, TAIL)], e2)
        pltpu.sync_copy(tgt_hbm.at[pl.ds(MAIN, TAIL)], et)
        a2 = zero16
        for i in range(TAIL // 16):
            a2 = a2 + sample16(i, ec, e0, e1, e2, et)
        acc[...] = a2

    for g in range(4):
        vc = zero16
        vs = zero16
        vr = zero16
        for l in range(16):
            o = l * NBINS + g * 16
            vs = vs + hs[pl.ds(o, 16)]
            packed = hr[pl.ds(o, 16)]
            cnt = (packed * (1.0 / 4096.0)).astype(jnp.int32).astype(jnp.float32)
            vc = vc + cnt
            vr = vr + (packed - cnt * 4096.0)
        prow[pl.ds(g * 16, 16)] = vc
        prow[pl.ds(64 + g * 16, 16)] = vs
        prow[pl.ds(128 + g * 16, 16)] = vr
    prow[pl.ds(192, 16)] = acc_main + acc[...]
    prow[pl.ds(208, 16)] = zero16
    prow[pl.ds(224, 16)] = zero16
    prow[pl.ds(240, 16)] = zero16
    pltpu.sync_copy(prow, out_hbm.at[pl.ds(wid * ROW, ROW)])


_sc_hist = functools.partial(
    pl.kernel,
    out_type=jax.ShapeDtypeStruct((NW * ROW,), jnp.float32),
    mesh=plsc.VectorSubcoreMesh(core_axis_name="c", subcore_axis_name="s"),
    compiler_params=pltpu.CompilerParams(needs_layout_passes=False),
    scratch_types=[
        pltpu.VMEM((CHUNK,), jnp.float32),
        pltpu.VMEM((CHUNK,), jnp.float32),
        pltpu.VMEM((CHUNK,), jnp.float32),
        pltpu.VMEM((CHUNK,), jnp.float32),
        pltpu.VMEM((CHUNK,), jnp.int32),
        pltpu.VMEM((CHUNK,), jnp.float32),
        pltpu.VMEM((CHUNK,), jnp.float32),
        pltpu.VMEM((CHUNK,), jnp.float32),
        pltpu.VMEM((CHUNK,), jnp.float32),
        pltpu.VMEM((CHUNK,), jnp.int32),
        pltpu.VMEM((TAIL,), jnp.float32),
        pltpu.VMEM((TAIL,), jnp.float32),
        pltpu.VMEM((TAIL,), jnp.float32),
        pltpu.VMEM((TAIL,), jnp.float32),
        pltpu.VMEM((TAIL,), jnp.int32),
        pltpu.VMEM((56,), jnp.float32),
        pltpu.VMEM((16 * NBINS,), jnp.float32),
        pltpu.VMEM((16 * NBINS,), jnp.float32),
        pltpu.VMEM((16,), jnp.float32),
        pltpu.VMEM((ROW,), jnp.float32),
        pltpu.SemaphoreType.DMA,
        pltpu.SemaphoreType.DMA,
    ],
)(_sc_body)


def _combine(x_ref, t_ref, b_ref, e_ref):
    x = x_ref[...]
    nf = jnp.float32(N)
    cnt = jnp.sum(x[:, 0:64], axis=0, keepdims=True)
    sconf = jnp.sum(x[:, 64:128], axis=0, keepdims=True)
    scorr = jnp.sum(x[:, 128:192], axis=0, keepdims=True)
    bce = jnp.sum(x[:, 192:208]) / nf
    safe = jnp.maximum(cnt, 1.0)
    term = jnp.where(cnt > 0, (cnt / nf) * jnp.abs(scorr / safe - sconf / safe), 0.0)
    ece = jnp.sum(term)
    t_ref[0, 0] = bce + ece
    b_ref[0, 0] = bce
    e_ref[0, 0] = ece


def kernel(confidence, direction_logits, targets):
    conf = confidence.T
    lflat = direction_logits.T.reshape(3 * N)
    bounds = jnp.pad(jnp.linspace(0.0, 1.0, 51), (0, 5), constant_values=2.0)
    partial = _sc_hist(conf, lflat, targets, bounds)
    total, bce, ece = pl.pallas_call(
        _combine,
        out_shape=(
            jax.ShapeDtypeStruct((1, 1), jnp.float32),
            jax.ShapeDtypeStruct((1, 1), jnp.float32),
            jax.ShapeDtypeStruct((1, 1), jnp.float32),
        ),
        out_specs=(
            pl.BlockSpec(memory_space=pltpu.SMEM),
            pl.BlockSpec(memory_space=pltpu.SMEM),
            pl.BlockSpec(memory_space=pltpu.SMEM),
        ),
    )(partial.reshape(NW, ROW))
    return (total[0, 0], bce[0, 0], ece[0, 0])


# speedup vs baseline: 1.4984x; 1.2959x over previous
import functools

import jax
import jax.numpy as jnp
from jax import lax
from jax.experimental import pallas as pl
from jax.experimental.pallas import tpu as pltpu
from jax.experimental.pallas import tpu_sc as plsc

N = 1_000_000
NW = 32
PER_W = 31_232
MAIN = NW * PER_W
TAIL = N - MAIN
KCH = 4
CHUNK = PER_W // KCH
UNROLL = 4
IN_IT = CHUNK // 16
NBINS = 64
ROW = 256
LN2 = 0.69314718055994530942
SQRT2 = 1.4142135623730951


def _vlog(x):
    bits = plsc.bitcast(x, jnp.int32)
    e = (bits >> 23) - 127
    m = plsc.bitcast((bits & 0x007FFFFF) | 0x3F800000, jnp.float32)
    ef = e.astype(jnp.float32)
    s = (m - 1.0) / (m + 1.0)
    t = s * s
    poly = 1.0 / 3.0 + t * (1.0 / 5.0 + t * (1.0 / 7.0 + t * (1.0 / 9.0)))
    return ef * LN2 + 2.0 * s * (1.0 + t * poly)


def _sc_body(conf_hbm, lf_hbm, tgt_hbm, out_hbm,
             cbufA, b0A, b1A, b2A, tbufA, cbufB, b0B, b1B, b2B, tbufB,
             ec, e0, e1, e2, et, hs, hr, acc, prow, semA, semB):
    nc = 2
    wid = lax.axis_index("s") * nc + lax.axis_index("c")
    base = wid * PER_W
    lane = lax.iota(jnp.int32, 16)
    zero16 = jnp.zeros((16,), jnp.float32)

    for i in range(NBINS):
        hs[pl.ds(i * 16, 16)] = zero16
        hr[pl.ds(i * 16, 16)] = zero16

    def sample16(i, cb, lb0, lb1, lb2, tb):
        conf = cb[pl.ds(i * 16, 16)]
        tgt = tb[pl.ds(i * 16, 16)]
        l0 = lb0[pl.ds(i * 16, 16)]
        l1 = lb1[pl.ds(i * 16, 16)]
        l2 = lb2[pl.ds(i * 16, 16)]
        pred = jnp.where(l2 > jnp.maximum(l0, l1), 2, jnp.where(l1 > l0, 1, 0))
        corr = (pred == tgt).astype(jnp.float32)
        p = jnp.maximum(conf, 1e-12)
        q = jnp.where(corr > 0.5, p, 1.0 - p)
        y = conf * 50.0
        iy = y.astype(jnp.int32)
        j = iy + (y > iy.astype(jnp.float32)).astype(jnp.int32) - 1
        valid = j >= 0
        hidx = lane * NBINS + jnp.maximum(j, 0)
        plsc.addupdate_scatter(hs, [hidx], conf, mask=valid)
        plsc.addupdate_scatter(hr, [hidx], corr + 4096.0, mask=valid)
        return -_vlog(q)

    bufs = [(cbufA, b0A, b1A, b2A, tbufA, semA), (cbufB, b0B, b1B, b2B, tbufB, semB)]

    def start_chunk(g):
        cb, x0, x1, x2, tb, sem = bufs[g % 2]
        off = base + g * CHUNK
        copies = (
            pltpu.async_copy(conf_hbm.at[0, pl.ds(off, CHUNK)], cb, sem),
            pltpu.async_copy(lf_hbm.at[pl.ds(off, CHUNK)], x0, sem),
            pltpu.async_copy(lf_hbm.at[pl.ds(N + off, CHUNK)], x1, sem),
            pltpu.async_copy(lf_hbm.at[pl.ds(2 * N + off, CHUNK)], x2, sem),
            pltpu.async_copy(tgt_hbm.at[pl.ds(off, CHUNK)], tb, sem),
        )
        return copies

    def compute_chunk(copies, g, a):
        cb, x0, x1, x2, tb, sem = bufs[g % 2]
        for cp in copies:
            cp.wait()

        def inner(i, a2):
            for u in range(UNROLL):
                a2 = a2 + sample16(i * UNROLL + u, cb, x0, x1, x2, tb)
            return a2

        return lax.fori_loop(0, IN_IT // UNROLL, inner, a)

    acc_main = zero16
    pending = start_chunk(0)
    for g in range(KCH):
        nxt = start_chunk(g + 1) if g + 1 < KCH else None
        acc_main = compute_chunk(pending, g, acc_main)
        pending = nxt
    acc[...] = zero16

    @pl.when(wid == 0)
    def _tail():
        pltpu.sync_copy(conf_hbm.at[0, pl.ds(MAIN, TAIL)], ec)
        pltpu.sync_copy(lf_hbm.at[pl.ds(MAIN, TAIL)], e0)
        pltpu.sync_copy(lf_hbm.at[pl.ds(N + MAIN, TAIL)], e1)
        pltpu.sync_copy(lf_hbm.at[pl.ds(2 * N + MAIN, TAIL)], e2)
        pltpu.sync_copy(tgt_hbm.at[pl.ds(MAIN, TAIL)], et)
        a2 = zero16
        for i in range(TAIL // 16):
            a2 = a2 + sample16(i, ec, e0, e1, e2, et)
        acc[...] = a2

    for g in range(4):
        vc = zero16
        vs = zero16
        vr = zero16
        for l in range(16):
            o = l * NBINS + g * 16
            vs = vs + hs[pl.ds(o, 16)]
            packed = hr[pl.ds(o, 16)]
            cnt = (packed * (1.0 / 4096.0)).astype(jnp.int32).astype(jnp.float32)
            vc = vc + cnt
            vr = vr + (packed - cnt * 4096.0)
        prow[pl.ds(g * 16, 16)] = vc
        prow[pl.ds(64 + g * 16, 16)] = vs
        prow[pl.ds(128 + g * 16, 16)] = vr
    prow[pl.ds(192, 16)] = acc_main + acc[...]
    prow[pl.ds(208, 16)] = zero16
    prow[pl.ds(224, 16)] = zero16
    prow[pl.ds(240, 16)] = zero16
    pltpu.sync_copy(prow, out_hbm.at[pl.ds(wid * ROW, ROW)])


_sc_hist = functools.partial(
    pl.kernel,
    out_type=jax.ShapeDtypeStruct((NW * ROW,), jnp.float32),
    mesh=plsc.VectorSubcoreMesh(core_axis_name="c", subcore_axis_name="s"),
    compiler_params=pltpu.CompilerParams(needs_layout_passes=False),
    scratch_types=[
        pltpu.VMEM((CHUNK,), jnp.float32),
        pltpu.VMEM((CHUNK,), jnp.float32),
        pltpu.VMEM((CHUNK,), jnp.float32),
        pltpu.VMEM((CHUNK,), jnp.float32),
        pltpu.VMEM((CHUNK,), jnp.int32),
        pltpu.VMEM((CHUNK,), jnp.float32),
        pltpu.VMEM((CHUNK,), jnp.float32),
        pltpu.VMEM((CHUNK,), jnp.float32),
        pltpu.VMEM((CHUNK,), jnp.float32),
        pltpu.VMEM((CHUNK,), jnp.int32),
        pltpu.VMEM((TAIL,), jnp.float32),
        pltpu.VMEM((TAIL,), jnp.float32),
        pltpu.VMEM((TAIL,), jnp.float32),
        pltpu.VMEM((TAIL,), jnp.float32),
        pltpu.VMEM((TAIL,), jnp.int32),
        pltpu.VMEM((16 * NBINS,), jnp.float32),
        pltpu.VMEM((16 * NBINS,), jnp.float32),
        pltpu.VMEM((16,), jnp.float32),
        pltpu.VMEM((ROW,), jnp.float32),
        pltpu.SemaphoreType.DMA,
        pltpu.SemaphoreType.DMA,
    ],
)(_sc_body)


def _combine(x_ref, t_ref, b_ref, e_ref):
    x = x_ref[...]
    nf = jnp.float32(N)
    cnt = jnp.sum(x[:, 0:64], axis=0, keepdims=True)
    sconf = jnp.sum(x[:, 64:128], axis=0, keepdims=True)
    scorr = jnp.sum(x[:, 128:192], axis=0, keepdims=True)
    bce = jnp.sum(x[:, 192:208]) / nf
    safe = jnp.maximum(cnt, 1.0)
    term = jnp.where(cnt > 0, (cnt / nf) * jnp.abs(scorr / safe - sconf / safe), 0.0)
    ece = jnp.sum(term)
    t_ref[0, 0] = bce + ece
    b_ref[0, 0] = bce
    e_ref[0, 0] = ece


def kernel(confidence, direction_logits, targets):
    conf = confidence.T
    lflat = direction_logits.T.reshape(3 * N)
    partial = _sc_hist(conf, lflat, targets)
    total, bce, ece = pl.pallas_call(
        _combine,
        out_shape=(
            jax.ShapeDtypeStruct((1, 1), jnp.float32),
            jax.ShapeDtypeStruct((1, 1), jnp.float32),
            jax.ShapeDtypeStruct((1, 1), jnp.float32),
        ),
        out_specs=(
            pl.BlockSpec(memory_space=pltpu.SMEM),
            pl.BlockSpec(memory_space=pltpu.SMEM),
            pl.BlockSpec(memory_space=pltpu.SMEM),
        ),
    )(partial.reshape(NW, ROW))
    return (total[0, 0], bce[0, 0], ece[0, 0])
